# baseline (speedup 1.0000x reference)
import dataclasses
import functools
import math

import jax
import jax.numpy as jnp
from jax import lax
from jax.experimental import pallas as pl
from jax.experimental.pallas import tpu as pltpu
from jax.experimental.pallas import tpu_sc as plsc

NC = 2
NS = 16
B = 128
ZR = 128
N_EXTRA = 112


def _attn(tab, vs, vd):
  c = tab.shape[1]
  asrc = jnp.dot(tab, vs.reshape(c, 1), preferred_element_type=jnp.float32)
  adst = jnp.dot(tab, vd.reshape(c, 1), preferred_element_type=jnp.float32)
  m = jnp.maximum(jnp.max(asrc) + jnp.max(adst), 0.0)
  return asrc, adst, jnp.full((1, 16), m)


def _finish(acc, den, b, W=None):
  n = acc.shape[1] - N_EXTRA
  num = acc[0, :n, :] + acc[1, :n, :]
  if W is not None:
    num = jnp.dot(num, W, preferred_element_type=jnp.float32)
  d = den[0, :n, :] + den[1, :n, :]
  return num / (d + 1e-16) + b


def _elu(x):
  return jnp.where(x > 0, x, jnp.exp(jnp.minimum(x, 0.0)) - 1.0)


def _tc_first_body(x_ref, W_ref, aS_ref, aD_ref,
                   tab_ref, als_ref, ald_ref, m_ref):
  h = jnp.dot(x_ref[...], W_ref[...], preferred_element_type=jnp.float32)
  als, ald, m = _attn(h, aS_ref[...], aD_ref[...])
  tab_ref[...] = h
  als_ref[...] = als
  ald_ref[...] = ald
  m_ref[...] = m


def _tc_mid1_body(acc_ref, den_ref, b_ref, Wn_ref, aS_ref, aD_ref,
                  tab_ref, als_ref, ald_ref, m_ref):
  xn = _elu(_finish(acc_ref[...], den_ref[...], b_ref[...]))
  Wn = Wn_ref[...]
  als, ald, m = _attn(xn, jnp.dot(Wn, aS_ref[...].reshape(-1, 1)),
                      jnp.dot(Wn, aD_ref[...].reshape(-1, 1)))
  tab_ref[...] = xn
  als_ref[...] = als
  ald_ref[...] = ald
  m_ref[...] = m


def _tc_mid2_body(acc_ref, den_ref, b_ref, Wp_ref, Wn_ref, aS_ref, aD_ref,
                  tab_ref, als_ref, ald_ref, m_ref):
  xn = _elu(_finish(acc_ref[...], den_ref[...], b_ref[...], Wp_ref[...]))
  Wn = Wn_ref[...]
  als, ald, m = _attn(xn, jnp.dot(Wn, aS_ref[...].reshape(-1, 1)),
                      jnp.dot(Wn, aD_ref[...].reshape(-1, 1)))
  tab_ref[...] = xn
  als_ref[...] = als
  ald_ref[...] = ald
  m_ref[...] = m


def _tc_fin_body(acc_ref, den_ref, b_ref, Wp_ref, out_ref):
  out_ref[...] = _finish(acc_ref[...], den_ref[...], b_ref[...], Wp_ref[...])


@functools.lru_cache(maxsize=None)
def _make_sc_edge(N, C, E_tot, chunks):
  mesh = plsc.VectorSubcoreMesh(core_axis_name="c", subcore_axis_name="s")
  cp = pltpu.CompilerParams()
  if "needs_layout_passes" in pltpu.CompilerParams.__dataclass_fields__:
    cp = dataclasses.replace(cp, needs_layout_passes=False)
  if "use_tc_tiling_on_sc" in pltpu.CompilerParams.__dataclass_fields__:
    cp = dataclasses.replace(cp, use_tc_tiling_on_sc=False)
  NPAD = N + N_EXTRA
  RPT = NPAD // NS
  RPTZ = ((RPT + 15) // 16) * 16

  @functools.partial(
      pl.kernel,
      out_type=[jax.ShapeDtypeStruct((NC, NPAD, C), jnp.float32),
                jax.ShapeDtypeStruct((NC, NPAD), jnp.float32)],
      mesh=mesh,
      compiler_params=cp,
      scratch_types=[
          pltpu.VMEM((N,), jnp.float32),
          pltpu.VMEM((N,), jnp.float32),
          pltpu.VMEM((16,), jnp.float32),
          pltpu.VMEM((chunks, B), jnp.int32),
          pltpu.VMEM((chunks, B), jnp.int32),
          pltpu.VMEM((B,), jnp.float32),
          pltpu.VMEM((B,), jnp.float32),
          pltpu.VMEM((B, C), jnp.float32),
          pltpu.VMEM((B, C), jnp.float32),
          pltpu.VMEM((ZR, C), jnp.float32),
          pltpu.VMEM((RPTZ,), jnp.float32),
          pltpu.VMEM_SHARED((NPAD, C), jnp.float32),
          pltpu.VMEM_SHARED((NPAD,), jnp.float32),
          pltpu.SemaphoreType.DMA,
          pltpu.SemaphoreType.DMA,
          pltpu.SemaphoreType.DMA,
          pltpu.SemaphoreType.DMA,
          pltpu.SemaphoreType.DMA,
          pltpu.SemaphoreType.DMA,
      ],
  )
  def sc_edge(hpad_hbm, als_hbm, ald_hbm, m_hbm, src_hbm, dst_hbm,
              acc_hbm, den_hbm,
              as_v, ad_v, m_v, sall_v, dall_v, w0_v, w1_v, rows0_v, rows1_v,
              zero_v, zden_v, acc_sh, den_sh,
              gsem0, gsem1, ssem0, ssem1, dsem0, dsem1):
    cid = lax.axis_index("c")
    sid = lax.axis_index("s")
    wid = cid * NS + sid

    @pl.loop(0, ZR)
    def _(r):
      for cc in range(C // 16):
        zero_v[r, pl.ds(cc * 16, 16)] = jnp.zeros((16,), jnp.float32)

    @pl.loop(0, RPTZ, step=16)
    def _(r):
      zden_v[pl.ds(r, 16)] = jnp.zeros((16,), jnp.float32)

    staged = []
    off = 0
    while off < RPT:
      sz = min(ZR, RPT - off)
      staged.append(pltpu.async_copy(
          zero_v.at[pl.ds(0, sz)], acc_sh.at[pl.ds(sid * RPT + off, sz)],
          gsem0))
      off += sz
    staged.append(pltpu.async_copy(
        zden_v.at[pl.ds(0, RPT)], den_sh.at[pl.ds(sid * RPT, RPT)], gsem0))
    staged.append(pltpu.async_copy(
        src_hbm.at[pl.ds(wid * chunks, chunks)], sall_v, ssem0))
    staged.append(pltpu.async_copy(
        dst_hbm.at[pl.ds(wid * chunks, chunks)], dall_v, ssem0))
    staged.append(pltpu.async_copy(als_hbm.at[0], as_v, dsem0))
    staged.append(pltpu.async_copy(ald_hbm.at[0], ad_v, dsem0))
    staged.append(pltpu.async_copy(m_hbm.at[0], m_v, dsem0))
    for d in staged:
      d.wait()
    plsc.subcore_barrier()

    m16 = m_v[...]
    base0 = wid * (chunks * B)

    def compute_w(ch, w_ref):
      @pl.loop(0, B, step=16)
      def _(j):
        s16 = sall_v[ch, pl.ds(j, 16)]
        d16 = dall_v[ch, pl.ds(j, 16)]
        z = plsc.load_gather(as_v, [s16]) + plsc.load_gather(ad_v, [d16])
        e = jnp.where(z >= 0, z, 0.2 * z)
        w = jnp.exp(e - m16)
        pos = base0 + ch * B + j + lax.iota(jnp.int32, 16)
        w_ref[pl.ds(j, 16)] = jnp.where(pos < E_tot, w, 0.0)

    def scale(rows_ref, w_ref):
      @pl.loop(0, B, step=16)
      def _(j):
        w16 = w_ref[pl.ds(j, 16)]
        for ii in range(16):
          ws = w16[ii]
          for cc in range(C // 16):
            sl = pl.ds(cc * 16, 16)
            rows_ref[j + ii, sl] = rows_ref[j + ii, sl] * ws

    def issue_gather(ch, rows_ref, sem):
      return pltpu.async_copy(hpad_hbm.at[sall_v.at[ch]], rows_ref, sem)

    def wait_gather(rows_ref, sem):
      pltpu.make_async_copy(hpad_hbm.at[sall_v.at[0]], rows_ref, sem).wait()

    def issue_scat(ch, rows_ref, sem):
      return pltpu.async_copy(rows_ref, acc_sh.at[dall_v.at[ch]], sem,
                              add=True)

    def wait_scat(rows_ref, sem):
      pltpu.make_async_copy(rows_ref, acc_sh.at[dall_v.at[0]], sem).wait()

    def issue_den(ch, w_ref, sem):
      return pltpu.async_copy(w_ref, den_sh.at[dall_v.at[ch]], sem, add=True)

    def wait_den(w_ref, sem):
      pltpu.make_async_copy(w_ref, den_sh.at[dall_v.at[0]], sem).wait()

    g0 = issue_gather(0, rows0_v, gsem0)
    compute_w(0, w0_v)
    g0.wait()
    issue_gather(1, rows1_v, gsem1)
    compute_w(1, w1_v)
    scale(rows0_v, w0_v)
    issue_scat(0, rows0_v, ssem0)
    issue_den(0, w0_v, dsem0)

    @pl.loop(0, (chunks - 2) // 2)
    def _(g):
      ca = 2 * g + 1
      wait_gather(rows1_v, gsem1)
      wait_scat(rows0_v, ssem0)
      wait_den(w0_v, dsem0)
      gb = issue_gather(ca + 1, rows0_v, gsem0)
      compute_w(ca + 1, w0_v)
      scale(rows1_v, w1_v)
      sa = issue_scat(ca, rows1_v, ssem1)
      da = issue_den(ca, w1_v, dsem1)
      gb.wait()
      sa.wait()
      da.wait()
      issue_gather(ca + 2, rows1_v, gsem1)
      compute_w(ca + 2, w1_v)
      scale(rows0_v, w0_v)
      issue_scat(ca + 1, rows0_v, ssem0)
      issue_den(ca + 1, w0_v, dsem0)

    wait_gather(rows1_v, gsem1)
    wait_scat(rows0_v, ssem0)
    wait_den(w0_v, dsem0)
    scale(rows1_v, w1_v)
    pltpu.sync_copy(rows1_v, acc_sh.at[dall_v.at[chunks - 1]], add=True)
    pltpu.sync_copy(w1_v, den_sh.at[dall_v.at[chunks - 1]], add=True)

    plsc.subcore_barrier()
    off = 0
    while off < RPT:
      sz = min(ZR, RPT - off)
      pltpu.sync_copy(acc_sh.at[pl.ds(sid * RPT + off, sz)],
                      acc_hbm.at[cid, pl.ds(sid * RPT + off, sz)])
      off += sz
    pltpu.sync_copy(den_sh.at[pl.ds(sid * RPT, RPT)],
                    den_hbm.at[cid, pl.ds(sid * RPT, RPT)])

  return sc_edge


def kernel(x, edge_index, W1, aS1, aD1, b1, W2, aS2, aD2, b2,
           W3, aS3, aD3, b3):
  N, _ = x.shape
  E = edge_index.shape[1]
  C_hid = W1.shape[1]
  C_out = W3.shape[1]

  src = edge_index[0].astype(jnp.int32)
  dst = edge_index[1].astype(jnp.int32)
  loop_idx = jnp.arange(N, dtype=jnp.int32)
  E_tot = E + N
  chunks = 2 * math.ceil(E_tot / (2 * NC * NS * B))
  EP = NC * NS * B * chunks
  pad = jnp.zeros((EP - E_tot,), jnp.int32)
  srcp = jnp.concatenate([src, loop_idx, pad]).reshape(EP // B, B)
  dstp = jnp.concatenate([dst, loop_idx, pad]).reshape(EP // B, B)

  tab_shapes = [jax.ShapeDtypeStruct((N, C_hid), jnp.float32),
                jax.ShapeDtypeStruct((N, 1), jnp.float32),
                jax.ShapeDtypeStruct((N, 1), jnp.float32),
                jax.ShapeDtypeStruct((1, 16), jnp.float32)]
  npad = N + N_EXTRA
  r1 = lambda a: a.reshape(1, N)
  rd = lambda d: d.reshape(NC, npad, 1)
  tc_first = pl.pallas_call(_tc_first_body, out_shape=tab_shapes)
  tc_mid1 = pl.pallas_call(_tc_mid1_body, out_shape=tab_shapes)
  tc_mid2 = pl.pallas_call(_tc_mid2_body, out_shape=tab_shapes)
  tc_fin = pl.pallas_call(
      _tc_fin_body, out_shape=jax.ShapeDtypeStruct((N, C_out), jnp.float32))

  sc_h = _make_sc_edge(N, C_hid, E_tot, chunks)

  hp1, as1, ad1, m1 = tc_first(x, W1, aS1, aD1)
  acc1, den1 = sc_h(hp1, r1(as1), r1(ad1), m1, srcp, dstp)
  hp2, as2, ad2, m2 = tc_mid1(acc1, rd(den1), b1, W2, aS2, aD2)
  acc2, den2 = sc_h(hp2, r1(as2), r1(ad2), m2, srcp, dstp)
  hp3, as3, ad3, m3 = tc_mid2(acc2, rd(den2), b2, W2, W3, aS3, aD3)
  acc3, den3 = sc_h(hp3, r1(as3), r1(ad3), m3, srcp, dstp)
  return tc_fin(acc3, rd(den3), b3, W3)


# speedup vs baseline: 1.2253x; 1.2253x over previous
import dataclasses
import functools
import math

import jax
import jax.numpy as jnp
from jax import lax
from jax.experimental import pallas as pl
from jax.experimental.pallas import tpu as pltpu
from jax.experimental.pallas import tpu_sc as plsc

NC = 2
NS = 16
B = 128
ZR = 128
N_EXTRA = 112


def _attn(tab, vs, vd):
  c = tab.shape[1]
  dn = (((1,), (1,)), ((), ()))
  asrc = lax.dot_general(vs.reshape(1, c), tab, dn,
                         preferred_element_type=jnp.float32)
  adst = lax.dot_general(vd.reshape(1, c), tab, dn,
                         preferred_element_type=jnp.float32)
  m = jnp.maximum(jnp.max(asrc) + jnp.max(adst), 0.0)
  return asrc, adst, jnp.full((1, 16), m)


def _finish(acc, den, b, W=None):
  n = acc.shape[1] - N_EXTRA
  num = acc[0, :n, :] + acc[1, :n, :]
  if W is not None:
    num = jnp.dot(num, W, preferred_element_type=jnp.float32)
  d = (den[0:1, :n] + den[1:2, :n]).T
  return num / (d + 1e-16) + b


def _elu(x):
  return jnp.where(x > 0, x, jnp.exp(jnp.minimum(x, 0.0)) - 1.0)


def _tc_first_body(x_ref, W_ref, aS_ref, aD_ref,
                   tab_ref, als_ref, ald_ref, m_ref):
  h = jnp.dot(x_ref[...], W_ref[...], preferred_element_type=jnp.float32)
  als, ald, m = _attn(h, aS_ref[...], aD_ref[...])
  tab_ref[...] = h
  als_ref[...] = als
  ald_ref[...] = ald
  m_ref[...] = m


def _tc_mid1_body(acc_ref, den_ref, b_ref, Wn_ref, aS_ref, aD_ref,
                  tab_ref, als_ref, ald_ref, m_ref):
  xn = _elu(_finish(acc_ref[...], den_ref[...], b_ref[...]))
  Wn = Wn_ref[...]
  als, ald, m = _attn(xn, jnp.dot(Wn, aS_ref[...].reshape(-1, 1)),
                      jnp.dot(Wn, aD_ref[...].reshape(-1, 1)))
  tab_ref[...] = xn
  als_ref[...] = als
  ald_ref[...] = ald
  m_ref[...] = m


def _tc_mid2_body(acc_ref, den_ref, b_ref, Wp_ref, Wn_ref, aS_ref, aD_ref,
                  tab_ref, als_ref, ald_ref, m_ref):
  xn = _elu(_finish(acc_ref[...], den_ref[...], b_ref[...], Wp_ref[...]))
  Wn = Wn_ref[...]
  als, ald, m = _attn(xn, jnp.dot(Wn, aS_ref[...].reshape(-1, 1)),
                      jnp.dot(Wn, aD_ref[...].reshape(-1, 1)))
  tab_ref[...] = xn
  als_ref[...] = als
  ald_ref[...] = ald
  m_ref[...] = m


def _tc_fin_body(acc_ref, den_ref, b_ref, Wp_ref, out_ref):
  out_ref[...] = _finish(acc_ref[...], den_ref[...], b_ref[...], Wp_ref[...])


@functools.lru_cache(maxsize=None)
def _make_sc_edge(N, C, E_tot, chunks):
  mesh = plsc.VectorSubcoreMesh(core_axis_name="c", subcore_axis_name="s")
  cp = pltpu.CompilerParams()
  if "needs_layout_passes" in pltpu.CompilerParams.__dataclass_fields__:
    cp = dataclasses.replace(cp, needs_layout_passes=False)
  if "use_tc_tiling_on_sc" in pltpu.CompilerParams.__dataclass_fields__:
    cp = dataclasses.replace(cp, use_tc_tiling_on_sc=False)
  NPAD = N + N_EXTRA
  RPT = NPAD // NS
  RPTZ = ((RPT + 15) // 16) * 16

  @functools.partial(
      pl.kernel,
      out_type=[jax.ShapeDtypeStruct((NC, NPAD, C), jnp.float32),
                jax.ShapeDtypeStruct((NC, NPAD), jnp.float32)],
      mesh=mesh,
      compiler_params=cp,
      scratch_types=[
          pltpu.VMEM((N,), jnp.float32),
          pltpu.VMEM((N,), jnp.float32),
          pltpu.VMEM((16,), jnp.float32),
          pltpu.VMEM((chunks, B), jnp.int32),
          pltpu.VMEM((chunks, B), jnp.int32),
          pltpu.VMEM((B,), jnp.float32),
          pltpu.VMEM((B,), jnp.float32),
          pltpu.VMEM((B, C), jnp.float32),
          pltpu.VMEM((B, C), jnp.float32),
          pltpu.VMEM((ZR, C), jnp.float32),
          pltpu.VMEM((RPTZ,), jnp.float32),
          pltpu.VMEM_SHARED((NPAD, C), jnp.float32),
          pltpu.VMEM_SHARED((NPAD,), jnp.float32),
          pltpu.SemaphoreType.DMA,
          pltpu.SemaphoreType.DMA,
          pltpu.SemaphoreType.DMA,
          pltpu.SemaphoreType.DMA,
          pltpu.SemaphoreType.DMA,
          pltpu.SemaphoreType.DMA,
      ],
  )
  def sc_edge(hpad_hbm, als_hbm, ald_hbm, m_hbm, src_hbm, dst_hbm,
              acc_hbm, den_hbm,
              as_v, ad_v, m_v, sall_v, dall_v, w0_v, w1_v, rows0_v, rows1_v,
              zero_v, zden_v, acc_sh, den_sh,
              gsem0, gsem1, ssem0, ssem1, dsem0, dsem1):
    cid = lax.axis_index("c")
    sid = lax.axis_index("s")
    wid = cid * NS + sid

    @pl.loop(0, ZR)
    def _(r):
      for cc in range(C // 16):
        zero_v[r, pl.ds(cc * 16, 16)] = jnp.zeros((16,), jnp.float32)

    @pl.loop(0, RPTZ, step=16)
    def _(r):
      zden_v[pl.ds(r, 16)] = jnp.zeros((16,), jnp.float32)

    staged = []
    off = 0
    while off < RPT:
      sz = min(ZR, RPT - off)
      staged.append(pltpu.async_copy(
          zero_v.at[pl.ds(0, sz)], acc_sh.at[pl.ds(sid * RPT + off, sz)],
          gsem0))
      off += sz
    staged.append(pltpu.async_copy(
        zden_v.at[pl.ds(0, RPT)], den_sh.at[pl.ds(sid * RPT, RPT)], gsem0))
    staged.append(pltpu.async_copy(
        src_hbm.at[pl.ds(wid * chunks, chunks)], sall_v, ssem0))
    staged.append(pltpu.async_copy(
        dst_hbm.at[pl.ds(wid * chunks, chunks)], dall_v, ssem0))
    staged.append(pltpu.async_copy(als_hbm.at[0], as_v, dsem0))
    staged.append(pltpu.async_copy(ald_hbm.at[0], ad_v, dsem0))
    staged.append(pltpu.async_copy(m_hbm.at[0], m_v, dsem0))
    for d in staged:
      d.wait()
    plsc.subcore_barrier()

    m16 = m_v[...]
    base0 = wid * (chunks * B)

    def compute_w(ch, w_ref):
      @pl.loop(0, B, step=16)
      def _(j):
        s16 = sall_v[ch, pl.ds(j, 16)]
        d16 = dall_v[ch, pl.ds(j, 16)]
        z = plsc.load_gather(as_v, [s16]) + plsc.load_gather(ad_v, [d16])
        e = jnp.where(z >= 0, z, 0.2 * z)
        w = jnp.exp(e - m16)
        pos = base0 + ch * B + j + lax.iota(jnp.int32, 16)
        w_ref[pl.ds(j, 16)] = jnp.where(pos < E_tot, w, 0.0)

    def scale(rows_ref, w_ref):
      @pl.loop(0, B, step=16)
      def _(j):
        w16 = w_ref[pl.ds(j, 16)]
        for ii in range(16):
          ws = w16[ii]
          for cc in range(C // 16):
            sl = pl.ds(cc * 16, 16)
            rows_ref[j + ii, sl] = rows_ref[j + ii, sl] * ws

    def issue_gather(ch, rows_ref, sem):
      return pltpu.async_copy(hpad_hbm.at[sall_v.at[ch]], rows_ref, sem)

    def wait_gather(rows_ref, sem):
      pltpu.make_async_copy(hpad_hbm.at[sall_v.at[0]], rows_ref, sem).wait()

    def issue_scat(ch, rows_ref, sem):
      return pltpu.async_copy(rows_ref, acc_sh.at[dall_v.at[ch]], sem,
                              add=True)

    def wait_scat(rows_ref, sem):
      pltpu.make_async_copy(rows_ref, acc_sh.at[dall_v.at[0]], sem).wait()

    def issue_den(ch, w_ref, sem):
      return pltpu.async_copy(w_ref, den_sh.at[dall_v.at[ch]], sem, add=True)

    def wait_den(w_ref, sem):
      pltpu.make_async_copy(w_ref, den_sh.at[dall_v.at[0]], sem).wait()

    g0 = issue_gather(0, rows0_v, gsem0)
    compute_w(0, w0_v)
    g0.wait()
    issue_gather(1, rows1_v, gsem1)
    compute_w(1, w1_v)
    scale(rows0_v, w0_v)
    issue_scat(0, rows0_v, ssem0)
    issue_den(0, w0_v, dsem0)

    @pl.loop(0, (chunks - 2) // 2)
    def _(g):
      ca = 2 * g + 1
      wait_gather(rows1_v, gsem1)
      wait_scat(rows0_v, ssem0)
      wait_den(w0_v, dsem0)
      gb = issue_gather(ca + 1, rows0_v, gsem0)
      compute_w(ca + 1, w0_v)
      scale(rows1_v, w1_v)
      sa = issue_scat(ca, rows1_v, ssem1)
      da = issue_den(ca, w1_v, dsem1)
      gb.wait()
      sa.wait()
      da.wait()
      issue_gather(ca + 2, rows1_v, gsem1)
      compute_w(ca + 2, w1_v)
      scale(rows0_v, w0_v)
      issue_scat(ca + 1, rows0_v, ssem0)
      issue_den(ca + 1, w0_v, dsem0)

    wait_gather(rows1_v, gsem1)
    wait_scat(rows0_v, ssem0)
    wait_den(w0_v, dsem0)
    scale(rows1_v, w1_v)
    pltpu.sync_copy(rows1_v, acc_sh.at[dall_v.at[chunks - 1]], add=True)
    pltpu.sync_copy(w1_v, den_sh.at[dall_v.at[chunks - 1]], add=True)

    plsc.subcore_barrier()
    off = 0
    while off < RPT:
      sz = min(ZR, RPT - off)
      pltpu.sync_copy(acc_sh.at[pl.ds(sid * RPT + off, sz)],
                      acc_hbm.at[cid, pl.ds(sid * RPT + off, sz)])
      off += sz
    pltpu.sync_copy(den_sh.at[pl.ds(sid * RPT, RPT)],
                    den_hbm.at[cid, pl.ds(sid * RPT, RPT)])

  return sc_edge


def kernel(x, edge_index, W1, aS1, aD1, b1, W2, aS2, aD2, b2,
           W3, aS3, aD3, b3):
  N, _ = x.shape
  E = edge_index.shape[1]
  C_hid = W1.shape[1]
  C_out = W3.shape[1]

  src = edge_index[0].astype(jnp.int32)
  dst = edge_index[1].astype(jnp.int32)
  loop_idx = jnp.arange(N, dtype=jnp.int32)
  E_tot = E + N
  chunks = 2 * math.ceil(E_tot / (2 * NC * NS * B))
  EP = NC * NS * B * chunks
  pad = jnp.zeros((EP - E_tot,), jnp.int32)
  srcp = jnp.concatenate([src, loop_idx, pad]).reshape(EP // B, B)
  dstp = jnp.concatenate([dst, loop_idx, pad]).reshape(EP // B, B)

  tab_shapes = [jax.ShapeDtypeStruct((N, C_hid), jnp.float32),
                jax.ShapeDtypeStruct((1, N), jnp.float32),
                jax.ShapeDtypeStruct((1, N), jnp.float32),
                jax.ShapeDtypeStruct((1, 16), jnp.float32)]
  tc_first = pl.pallas_call(_tc_first_body, out_shape=tab_shapes)
  tc_mid1 = pl.pallas_call(_tc_mid1_body, out_shape=tab_shapes)
  tc_mid2 = pl.pallas_call(_tc_mid2_body, out_shape=tab_shapes)
  tc_fin = pl.pallas_call(
      _tc_fin_body, out_shape=jax.ShapeDtypeStruct((N, C_out), jnp.float32))

  sc_h = _make_sc_edge(N, C_hid, E_tot, chunks)

  hp1, as1, ad1, m1 = tc_first(x, W1, aS1, aD1)
  acc1, den1 = sc_h(hp1, as1, ad1, m1, srcp, dstp)
  hp2, as2, ad2, m2 = tc_mid1(acc1, den1, b1, W2, aS2, aD2)
  acc2, den2 = sc_h(hp2, as2, ad2, m2, srcp, dstp)
  hp3, as3, ad3, m3 = tc_mid2(acc2, den2, b2, W2, W3, aS3, aD3)
  acc3, den3 = sc_h(hp3, as3, ad3, m3, srcp, dstp)
  return tc_fin(acc3, den3, b3, W3)


# speedup vs baseline: 1.2488x; 1.0192x over previous
import dataclasses
import functools
import math

import jax
import jax.numpy as jnp
from jax import lax
from jax.experimental import pallas as pl
from jax.experimental.pallas import tpu as pltpu
from jax.experimental.pallas import tpu_sc as plsc

NC = 2
NS = 16
B = 128
ZR = 128
N_EXTRA = 112


def _attn(tab, vs, vd):
  c = tab.shape[1]
  dn = (((1,), (1,)), ((), ()))
  asrc = lax.dot_general(vs.reshape(1, c), tab, dn,
                         preferred_element_type=jnp.float32)
  adst = lax.dot_general(vd.reshape(1, c), tab, dn,
                         preferred_element_type=jnp.float32)
  m = jnp.maximum(jnp.max(asrc) + jnp.max(adst), 0.0)
  return asrc, adst, jnp.full((1, 16), m)


def _finish(acc, den, b, W=None):
  n = acc.shape[1] - N_EXTRA
  num = acc[0, :n, :] + acc[1, :n, :]
  if W is not None:
    num = jnp.dot(num, W, preferred_element_type=jnp.float32)
  d = (den[0:1, :n] + den[1:2, :n]).T
  return num / (d + 1e-16) + b


def _elu(x):
  return jnp.where(x > 0, x, jnp.exp(jnp.minimum(x, 0.0)) - 1.0)


def _tc_first_body(x_ref, W_ref, aS_ref, aD_ref,
                   tab_ref, als_ref, ald_ref, m_ref):
  h = jnp.dot(x_ref[...], W_ref[...], preferred_element_type=jnp.float32)
  als, ald, m = _attn(h, aS_ref[...], aD_ref[...])
  tab_ref[...] = h
  als_ref[...] = als
  ald_ref[...] = ald
  m_ref[...] = m


def _tc_mid1_body(acc_ref, den_ref, b_ref, Wn_ref, aS_ref, aD_ref,
                  tab_ref, als_ref, ald_ref, m_ref):
  xn = _elu(_finish(acc_ref[...], den_ref[...], b_ref[...]))
  Wn = Wn_ref[...]
  als, ald, m = _attn(xn, jnp.dot(Wn, aS_ref[...].reshape(-1, 1)),
                      jnp.dot(Wn, aD_ref[...].reshape(-1, 1)))
  tab_ref[...] = xn
  als_ref[...] = als
  ald_ref[...] = ald
  m_ref[...] = m


def _tc_mid2_body(acc_ref, den_ref, b_ref, Wp_ref, Wn_ref, aS_ref, aD_ref,
                  tab_ref, als_ref, ald_ref, m_ref):
  xn = _elu(_finish(acc_ref[...], den_ref[...], b_ref[...], Wp_ref[...]))
  Wn = Wn_ref[...]
  als, ald, m = _attn(xn, jnp.dot(Wn, aS_ref[...].reshape(-1, 1)),
                      jnp.dot(Wn, aD_ref[...].reshape(-1, 1)))
  tab_ref[...] = xn
  als_ref[...] = als
  ald_ref[...] = ald
  m_ref[...] = m


def _tc_fin_body(acc_ref, den_ref, b_ref, Wp_ref, out_ref):
  out_ref[...] = _finish(acc_ref[...], den_ref[...], b_ref[...], Wp_ref[...])


@functools.lru_cache(maxsize=None)
def _make_sc_edge(N, C, E_tot, chunks):
  mesh = plsc.VectorSubcoreMesh(core_axis_name="c", subcore_axis_name="s")
  cp = pltpu.CompilerParams()
  if "needs_layout_passes" in pltpu.CompilerParams.__dataclass_fields__:
    cp = dataclasses.replace(cp, needs_layout_passes=False)
  if "use_tc_tiling_on_sc" in pltpu.CompilerParams.__dataclass_fields__:
    cp = dataclasses.replace(cp, use_tc_tiling_on_sc=False)
  NPAD = N + N_EXTRA
  RPT = NPAD // NS
  RPTZ = ((RPT + 15) // 16) * 16

  @functools.partial(
      pl.kernel,
      out_type=[jax.ShapeDtypeStruct((NC, NPAD, C), jnp.float32),
                jax.ShapeDtypeStruct((NC, NPAD), jnp.float32)],
      mesh=mesh,
      compiler_params=cp,
      scratch_types=[
          pltpu.VMEM((N,), jnp.float32),
          pltpu.VMEM((N,), jnp.float32),
          pltpu.VMEM((16,), jnp.float32),
          pltpu.VMEM((chunks, B), jnp.int32),
          pltpu.VMEM((chunks, B), jnp.int32),
          pltpu.VMEM((B,), jnp.float32),
          pltpu.VMEM((B,), jnp.float32),
          pltpu.VMEM((B, C), jnp.float32),
          pltpu.VMEM((B, C), jnp.float32),
          pltpu.VMEM((ZR, C), jnp.float32),
          pltpu.VMEM((RPTZ,), jnp.float32),
          pltpu.VMEM_SHARED((NPAD, C), jnp.float32),
          pltpu.VMEM_SHARED((NPAD,), jnp.float32),
          pltpu.SemaphoreType.DMA,
          pltpu.SemaphoreType.DMA,
          pltpu.SemaphoreType.DMA,
          pltpu.SemaphoreType.DMA,
          pltpu.SemaphoreType.DMA,
          pltpu.SemaphoreType.DMA,
      ],
  )
  def sc_edge(hpad_hbm, als_hbm, ald_hbm, m_hbm, src_hbm, dst_hbm,
              acc_hbm, den_hbm,
              as_v, ad_v, m_v, sall_v, dall_v, w0_v, w1_v, rows0_v, rows1_v,
              zero_v, zden_v, acc_sh, den_sh,
              gsem0, gsem1, ssem0, ssem1, dsem0, dsem1):
    cid = lax.axis_index("c")
    sid = lax.axis_index("s")
    wid = cid * NS + sid

    @pl.loop(0, ZR)
    def _(r):
      for cc in range(C // 16):
        zero_v[r, pl.ds(cc * 16, 16)] = jnp.zeros((16,), jnp.float32)

    @pl.loop(0, RPTZ, step=16)
    def _(r):
      zden_v[pl.ds(r, 16)] = jnp.zeros((16,), jnp.float32)

    staged = []
    off = 0
    while off < RPT:
      sz = min(ZR, RPT - off)
      staged.append(pltpu.async_copy(
          zero_v.at[pl.ds(0, sz)], acc_sh.at[pl.ds(sid * RPT + off, sz)],
          gsem0))
      off += sz
    staged.append(pltpu.async_copy(
        zden_v.at[pl.ds(0, RPT)], den_sh.at[pl.ds(sid * RPT, RPT)], gsem0))
    staged.append(pltpu.async_copy(
        src_hbm.at[pl.ds(wid * chunks, chunks)], sall_v, ssem0))
    staged.append(pltpu.async_copy(
        dst_hbm.at[pl.ds(wid * chunks, chunks)], dall_v, ssem0))
    staged.append(pltpu.async_copy(als_hbm.at[0], as_v, dsem0))
    staged.append(pltpu.async_copy(ald_hbm.at[0], ad_v, dsem0))
    staged.append(pltpu.async_copy(m_hbm.at[0], m_v, dsem0))
    for d in staged:
      d.wait()
    plsc.subcore_barrier()

    m16 = m_v[...]
    base0 = wid * (chunks * B)

    def compute_w(ch, w_ref):
      @pl.loop(0, B, step=16)
      def _(j):
        s16 = sall_v[ch, pl.ds(j, 16)]
        d16 = dall_v[ch, pl.ds(j, 16)]
        z = plsc.load_gather(as_v, [s16]) + plsc.load_gather(ad_v, [d16])
        e = jnp.where(z >= 0, z, 0.2 * z)
        w = jnp.exp(e - m16)
        pos = base0 + ch * B + j + lax.iota(jnp.int32, 16)
        w_ref[pl.ds(j, 16)] = jnp.where(pos < E_tot, w, 0.0)

    def scale(rows_ref, w_ref):
      @pl.loop(0, B, step=16)
      def _(j):
        w16 = w_ref[pl.ds(j, 16)]
        for ii in range(16):
          ws = w16[ii]
          for cc in range(C // 16):
            sl = pl.ds(cc * 16, 16)
            rows_ref[j + ii, sl] = rows_ref[j + ii, sl] * ws

    def issue_gather(ch, rows_ref, sem):
      return pltpu.async_copy(hpad_hbm.at[sall_v.at[ch]], rows_ref, sem)

    def wait_gather(rows_ref, sem):
      pltpu.make_async_copy(hpad_hbm.at[sall_v.at[0]], rows_ref, sem).wait()

    def issue_scat(ch, rows_ref, sem):
      return pltpu.async_copy(rows_ref, acc_sh.at[dall_v.at[ch]], sem,
                              add=True)

    def wait_scat(rows_ref, sem):
      pltpu.make_async_copy(rows_ref, acc_sh.at[dall_v.at[0]], sem).wait()

    def issue_den(ch, w_ref, sem):
      return pltpu.async_copy(w_ref, den_sh.at[dall_v.at[ch]], sem, add=True)

    def wait_den(w_ref, sem):
      pltpu.make_async_copy(w_ref, den_sh.at[dall_v.at[0]], sem).wait()

    g0 = issue_gather(0, rows0_v, gsem0)
    compute_w(0, w0_v)
    g0.wait()
    issue_gather(1, rows1_v, gsem1)
    compute_w(1, w1_v)
    scale(rows0_v, w0_v)
    issue_scat(0, rows0_v, ssem0)
    issue_den(0, w0_v, dsem0)

    @pl.loop(0, (chunks - 2) // 2)
    def _(g):
      ca = 2 * g + 1
      wait_gather(rows1_v, gsem1)
      wait_scat(rows0_v, ssem0)
      wait_den(w0_v, dsem0)
      gb = issue_gather(ca + 1, rows0_v, gsem0)
      compute_w(ca + 1, w0_v)
      scale(rows1_v, w1_v)
      sa = issue_scat(ca, rows1_v, ssem1)
      da = issue_den(ca, w1_v, dsem1)
      gb.wait()
      sa.wait()
      da.wait()
      issue_gather(ca + 2, rows1_v, gsem1)
      compute_w(ca + 2, w1_v)
      scale(rows0_v, w0_v)
      issue_scat(ca + 1, rows0_v, ssem0)
      issue_den(ca + 1, w0_v, dsem0)

    wait_gather(rows1_v, gsem1)
    wait_scat(rows0_v, ssem0)
    wait_den(w0_v, dsem0)
    scale(rows1_v, w1_v)
    pltpu.sync_copy(rows1_v, acc_sh.at[dall_v.at[chunks - 1]], add=True)
    pltpu.sync_copy(w1_v, den_sh.at[dall_v.at[chunks - 1]], add=True)

    plsc.subcore_barrier()
    dumps = []
    off = 0
    while off < RPT:
      sz = min(ZR, RPT - off)
      dumps.append(pltpu.async_copy(
          acc_sh.at[pl.ds(sid * RPT + off, sz)],
          acc_hbm.at[cid, pl.ds(sid * RPT + off, sz)], gsem0))
      off += sz
    dumps.append(pltpu.async_copy(
        den_sh.at[pl.ds(sid * RPT, RPT)],
        den_hbm.at[cid, pl.ds(sid * RPT, RPT)], ssem0))
    for d in dumps:
      d.wait()

  return sc_edge


def kernel(x, edge_index, W1, aS1, aD1, b1, W2, aS2, aD2, b2,
           W3, aS3, aD3, b3):
  N, _ = x.shape
  E = edge_index.shape[1]
  C_hid = W1.shape[1]
  C_out = W3.shape[1]

  src = edge_index[0].astype(jnp.int32)
  dst = edge_index[1].astype(jnp.int32)
  loop_idx = jnp.arange(N, dtype=jnp.int32)
  E_tot = E + N
  chunks = 2 * math.ceil(E_tot / (2 * NC * NS * B))
  EP = NC * NS * B * chunks
  pad = jnp.zeros((EP - E_tot,), jnp.int32)
  srcp = jnp.concatenate([src, loop_idx, pad]).reshape(EP // B, B)
  dstp = jnp.concatenate([dst, loop_idx, pad]).reshape(EP // B, B)

  tab_shapes = [jax.ShapeDtypeStruct((N, C_hid), jnp.float32),
                jax.ShapeDtypeStruct((1, N), jnp.float32),
                jax.ShapeDtypeStruct((1, N), jnp.float32),
                jax.ShapeDtypeStruct((1, 16), jnp.float32)]
  tc_first = pl.pallas_call(_tc_first_body, out_shape=tab_shapes)
  tc_mid1 = pl.pallas_call(_tc_mid1_body, out_shape=tab_shapes)
  tc_mid2 = pl.pallas_call(_tc_mid2_body, out_shape=tab_shapes)
  tc_fin = pl.pallas_call(
      _tc_fin_body, out_shape=jax.ShapeDtypeStruct((N, C_out), jnp.float32))

  sc_h = _make_sc_edge(N, C_hid, E_tot, chunks)

  hp1, as1, ad1, m1 = tc_first(x, W1, aS1, aD1)
  acc1, den1 = sc_h(hp1, as1, ad1, m1, srcp, dstp)
  hp2, as2, ad2, m2 = tc_mid1(acc1, den1, b1, W2, aS2, aD2)
  acc2, den2 = sc_h(hp2, as2, ad2, m2, srcp, dstp)
  hp3, as3, ad3, m3 = tc_mid2(acc2, den2, b2, W2, W3, aS3, aD3)
  acc3, den3 = sc_h(hp3, as3, ad3, m3, srcp, dstp)
  return tc_fin(acc3, den3, b3, W3)
